# baseline (speedup 1.0000x reference)
import functools

import jax
import jax.numpy as jnp
from jax import lax
from jax.experimental import pallas as pl
from jax.experimental.pallas import tpu as pltpu
from jax.experimental.pallas import tpu_sc as plsc

B = 4
S = 8192
D = 768
NSPANS = 2048

NC = 2
NS = 16
NW = NC * NS
SPANS_PER_W = B * NSPANS // NW
ROWS_PER_W = 2 * SPANS_PER_W
CHUNK = 8
NCHUNK = SPANS_PER_W // CHUNK
RING = 8
LANES = 16
W_PER_BATCH = NW // B

_mesh = plsc.VectorSubcoreMesh(core_axis_name="c", subcore_axis_name="s")


@functools.partial(
    pl.kernel,
    mesh=_mesh,
    out_type=jax.ShapeDtypeStruct((B, NSPANS, 2 * D), jnp.float32),
    scratch_types=(
        [pltpu.VMEM((ROWS_PER_W,), jnp.int32)]
        + [pltpu.VMEM((2 * CHUNK, D), jnp.float32) for _ in range(RING)]
        + [pltpu.SemaphoreType.DMA, pltpu.SemaphoreType.DMA]
    ),
)
def _span_gather(table_hbm, idx_hbm, out_hbm, idx_v, *rest):
    bufs = rest[:RING]
    gsem, wsem = rest[RING], rest[RING + 1]

    wid = lax.axis_index("s") * NC + lax.axis_index("c")
    b = wid // W_PER_BATCH
    s0 = (wid % W_PER_BATCH) * SPANS_PER_W

    pltpu.sync_copy(idx_hbm.at[pl.ds(wid * ROWS_PER_W, ROWS_PER_W)], idx_v)

    gcp = [None] * RING
    wS = [None] * RING
    wE = [None] * RING

    def start_gather(g, slot):
        gcp[slot] = pltpu.async_copy(
            table_hbm.at[idx_v.at[pl.ds(g * 2 * CHUNK, 2 * CHUNK)]],
            bufs[slot], gsem)

    for g in range(min(RING - 1, NCHUNK)):
        start_gather(g, g)
    for g in range(NCHUNK):
        slot = g % RING
        if g + RING - 1 < NCHUNK:
            nslot = (g + RING - 1) % RING
            if wS[nslot] is not None:
                wS[nslot].wait()
                wE[nslot].wait()
            start_gather(g + RING - 1, nslot)
        gcp[slot].wait()
        row = pl.ds(s0 + g * CHUNK, CHUNK)
        wS[slot] = pltpu.async_copy(
            bufs[slot].at[pl.ds(0, CHUNK)],
            out_hbm.at[b, row, pl.ds(0, D)], wsem)
        wE[slot] = pltpu.async_copy(
            bufs[slot].at[pl.ds(CHUNK, CHUNK)],
            out_hbm.at[b, row, pl.ds(D, D)], wsem)
    for slot in range(RING):
        if wS[slot] is not None:
            wS[slot].wait()
            wE[slot].wait()


def kernel(sequence_tensor, span_indices):
    table = sequence_tensor.reshape(B * S, D)
    offs = (jnp.arange(B, dtype=span_indices.dtype) * S).reshape(B, 1, 1)
    flat = (span_indices + offs).reshape(NW, NCHUNK, CHUNK, 2)
    idx = jnp.swapaxes(flat, 2, 3).reshape(-1)
    return _span_gather(table, idx)


# speedup vs baseline: 1.0366x; 1.0366x over previous
import functools

import jax
import jax.numpy as jnp
from jax import lax
from jax.experimental import pallas as pl
from jax.experimental.pallas import tpu as pltpu
from jax.experimental.pallas import tpu_sc as plsc

B = 4
S = 8192
D = 768
NSPANS = 2048

NC = 2
NS = 16
NW = NC * NS
SPANS_PER_W = B * NSPANS // NW
ROWS_PER_W = 2 * SPANS_PER_W
CHUNK = 32
NCHUNK = SPANS_PER_W // CHUNK
RING = 2
LANES = 16
W_PER_BATCH = NW // B

_mesh = plsc.VectorSubcoreMesh(core_axis_name="c", subcore_axis_name="s")


@functools.partial(
    pl.kernel,
    mesh=_mesh,
    out_type=jax.ShapeDtypeStruct((B, NSPANS, 2 * D), jnp.float32),
    scratch_types=(
        [pltpu.VMEM((ROWS_PER_W,), jnp.int32)]
        + [pltpu.VMEM((2 * CHUNK, D), jnp.float32) for _ in range(RING)]
        + [pltpu.SemaphoreType.DMA, pltpu.SemaphoreType.DMA]
    ),
)
def _span_gather(table_hbm, idx_hbm, out_hbm, idx_v, *rest):
    bufs = rest[:RING]
    gsem, wsem = rest[RING], rest[RING + 1]

    wid = lax.axis_index("s") * NC + lax.axis_index("c")
    b = wid // W_PER_BATCH
    s0 = (wid % W_PER_BATCH) * SPANS_PER_W

    pltpu.sync_copy(idx_hbm.at[pl.ds(wid * ROWS_PER_W, ROWS_PER_W)], idx_v)

    gcp = [None] * RING
    wS = [None] * RING
    wE = [None] * RING

    def start_gather(g, slot):
        gcp[slot] = pltpu.async_copy(
            table_hbm.at[idx_v.at[pl.ds(g * 2 * CHUNK, 2 * CHUNK)]],
            bufs[slot], gsem)

    for g in range(min(RING - 1, NCHUNK)):
        start_gather(g, g)
    for g in range(NCHUNK):
        slot = g % RING
        if g + RING - 1 < NCHUNK:
            nslot = (g + RING - 1) % RING
            if wS[nslot] is not None:
                wS[nslot].wait()
                wE[nslot].wait()
            start_gather(g + RING - 1, nslot)
        gcp[slot].wait()
        row = pl.ds(s0 + g * CHUNK, CHUNK)
        wS[slot] = pltpu.async_copy(
            bufs[slot].at[pl.ds(0, CHUNK)],
            out_hbm.at[b, row, pl.ds(0, D)], wsem)
        wE[slot] = pltpu.async_copy(
            bufs[slot].at[pl.ds(CHUNK, CHUNK)],
            out_hbm.at[b, row, pl.ds(D, D)], wsem)
    for slot in range(RING):
        if wS[slot] is not None:
            wS[slot].wait()
            wE[slot].wait()


def kernel(sequence_tensor, span_indices):
    table = sequence_tensor.reshape(B * S, D)
    offs = (jnp.arange(B, dtype=span_indices.dtype) * S).reshape(B, 1, 1)
    flat = (span_indices + offs).reshape(NW, NCHUNK, CHUNK, 2)
    idx = jnp.swapaxes(flat, 2, 3).reshape(-1)
    return _span_gather(table, idx)
